# baseline (speedup 1.0000x reference)
import dataclasses
import functools

import jax
import jax.numpy as jnp
from jax import lax
from jax.experimental import pallas as pl
from jax.experimental.pallas import tpu as pltpu
from jax.experimental.pallas import tpu_sc as plsc

D = 200
DP = 256
NC = 2
NS = 16
L = 16
NW = NC * NS
W = 32
CHUNK_OFFS = tuple(range(0, 13 * L, L))


def _pad_rows(x):
    n = x.shape[0]
    br = 5000 if n % 5000 == 0 else n

    def body(x_ref, o_ref):
        o_ref[...] = jnp.pad(x_ref[...], ((0, 0), (0, DP - D)))

    return pl.pallas_call(
        body,
        grid=(n // br,),
        in_specs=[pl.BlockSpec((br, D), lambda i: (i, 0))],
        out_specs=pl.BlockSpec((br, DP), lambda i: (i, 0)),
        out_shape=jax.ShapeDtypeStruct((n, DP), jnp.float32),
    )(x)


def _pad_rows2(a, b):
    n = a.shape[0]
    br = 5000 if n % 5000 == 0 else n

    def body(a_ref, b_ref, oa_ref, ob_ref):
        oa_ref[...] = jnp.pad(a_ref[...], ((0, 0), (0, DP - D)))
        ob_ref[...] = jnp.pad(b_ref[...], ((0, 0), (0, DP - D)))

    return pl.pallas_call(
        body,
        grid=(n // br,),
        in_specs=[pl.BlockSpec((br, D), lambda i: (i, 0)),
                  pl.BlockSpec((br, D), lambda i: (i, 0))],
        out_specs=[pl.BlockSpec((br, DP), lambda i: (i, 0)),
                   pl.BlockSpec((br, DP), lambda i: (i, 0))],
        out_shape=[jax.ShapeDtypeStruct((n, DP), jnp.float32),
                   jax.ShapeDtypeStruct((n, DP), jnp.float32)],
    )(a, b)


def _rsqrt(x):
    i = lax.bitcast_convert_type(x, jnp.int32)
    i = jnp.int32(0x5F3759DF) - (i >> 1)
    y = lax.bitcast_convert_type(i, jnp.float32)
    for _ in range(3):
        y = y * (jnp.float32(1.5) - jnp.float32(0.5) * x * y * y)
    return y


def _build_dist_kernel(tot):
    rpt = tot // NW
    ch = rpt // W
    mesh = plsc.VectorSubcoreMesh(core_axis_name="c", subcore_axis_name="s")
    cp = pltpu.CompilerParams()
    if "needs_layout_passes" in pltpu.CompilerParams.__dataclass_fields__:
        cp = dataclasses.replace(cp, needs_layout_passes=False)

    @functools.partial(
        pl.kernel,
        mesh=mesh,
        compiler_params=cp,
        out_type=jax.ShapeDtypeStruct((tot,), jnp.float32),
        scratch_types=[
            pltpu.VMEM((rpt,), jnp.int32),
            pltpu.VMEM((rpt,), jnp.int32),
            pltpu.VMEM((rpt,), jnp.int32),
            pltpu.VMEM((2, W, DP), jnp.float32),
            pltpu.VMEM((2, W, DP), jnp.float32),
            pltpu.VMEM((2, W, DP), jnp.float32),
            pltpu.VMEM((2, W, DP), jnp.float32),
            pltpu.VMEM((2, W, DP), jnp.float32),
            pltpu.VMEM((2, W, DP), jnp.float32),
            pltpu.VMEM((rpt,), jnp.float32),
            pltpu.SemaphoreType.DMA,
            pltpu.SemaphoreType.DMA,
        ],
    )
    def dist_kernel(ent_e_hbm, rel_e_hbm, ent_t_hbm, rel_t_hbm,
                    h_hbm, t_hbm, r_hbm, out_hbm,
                    hi, ti, ri, he, ht, te, tt, re, rt, res, sem0, sem1):
        wid = lax.axis_index("s") * NC + lax.axis_index("c")
        base = wid * rpt
        pltpu.sync_copy(h_hbm.at[pl.ds(base, rpt)], hi)
        pltpu.sync_copy(t_hbm.at[pl.ds(base, rpt)], ti)
        pltpu.sync_copy(r_hbm.at[pl.ds(base, rpt)], ri)

        def _issue(off, b, sem):
            return [
                pltpu.async_copy(
                    ent_e_hbm.at[hi.at[pl.ds(off, W)]], he.at[b], sem),
                pltpu.async_copy(
                    ent_t_hbm.at[hi.at[pl.ds(off, W)]], ht.at[b], sem),
                pltpu.async_copy(
                    ent_e_hbm.at[ti.at[pl.ds(off, W)]], te.at[b], sem),
                pltpu.async_copy(
                    ent_t_hbm.at[ti.at[pl.ds(off, W)]], tt.at[b], sem),
                pltpu.async_copy(
                    rel_e_hbm.at[ri.at[pl.ds(off, W)]], re.at[b], sem),
                pltpu.async_copy(
                    rel_t_hbm.at[ri.at[pl.ds(off, W)]], rt.at[b], sem),
            ]

        lane_ids = lax.iota(jnp.int32, L)

        def _compute(off, b):
            def _row(w, dacc):
                z = jnp.zeros((L,), jnp.float32)
                sh = st = ah = at_ = chv = ctv = qv = z
                for o in CHUNK_OFFS:
                    sl = (b, w, pl.ds(o, L))
                    hev, htv = he[sl], ht[sl]
                    tev, ttv = te[sl], tt[sl]
                    rtv = rt[sl]
                    sh = sh + hev * htv
                    st = st + tev * ttv
                    ah = ah + hev * hev
                    at_ = at_ + tev * tev
                    chv = chv + hev * rtv
                    ctv = ctv + tev * rtv
                    qv = qv + rtv * rtv
                s_h, s_t = jnp.sum(sh), jnp.sum(st)
                a_h, a_t = jnp.sum(ah), jnp.sum(at_)
                c_h, c_t = jnp.sum(chv), jnp.sum(ctv)
                q = jnp.sum(qv)

                two = jnp.float32(2.0)
                eps = jnp.float32(1e-12)
                nh = a_h + two * s_h * c_h + s_h * s_h * q
                nt = a_t + two * s_t * c_t + s_t * s_t * q
                ih = _rsqrt(jnp.maximum(nh, eps))
                it = _rsqrt(jnp.maximum(nt, eps))
                g = s_h * ih - s_t * it

                acc = z
                for o in CHUNK_OFFS:
                    sl = (b, w, pl.ds(o, L))
                    hev, tev = he[sl], te[sl]
                    rtv, rev = rt[sl], re[sl]
                    term = jnp.abs(ih * hev - it * tev + g * rtv + rev)
                    acc = acc + term
                return jnp.where(lane_ids == w % L, jnp.sum(acc), dacc)

            for sub in range(W // L):
                dvec = lax.fori_loop(
                    0, L,
                    lambda w, a, _s=sub: _row(jnp.int32(_s * L) + w, a),
                    jnp.zeros((L,), jnp.float32))
                res[pl.ds(off + sub * L, L)] = dvec

        @pl.loop(0, ch // 2)
        def _pair(p):
            off0 = pl.multiple_of(p * (2 * W), W)
            off1 = off0 + W
            d0 = _issue(off0, 0, sem0)
            d1 = _issue(off1, 1, sem1)
            for dma in d0:
                dma.wait()
            _compute(off0, 0)
            for dma in d1:
                dma.wait()
            _compute(off1, 1)

        pltpu.sync_copy(res, out_hbm.at[pl.ds(base, rpt)])

    return dist_kernel


def kernel(ent_emb, rel_emb, ent_transfer, rel_transfer,
           pos_h_id, pos_t_id, pos_r_id, neg_h_id, neg_t_id, neg_r_id):
    b = pos_h_id.shape[0]
    h_id = jnp.concatenate([pos_h_id, neg_h_id]).astype(jnp.int32)
    t_id = jnp.concatenate([pos_t_id, neg_t_id]).astype(jnp.int32)
    r_id = jnp.concatenate([pos_r_id, neg_r_id]).astype(jnp.int32)
    ent_emb_p, ent_transfer_p = _pad_rows2(ent_emb, ent_transfer)
    rel_emb_p, rel_transfer_p = _pad_rows2(rel_emb, rel_transfer)
    dist = _build_dist_kernel(2 * b)(
        ent_emb_p, rel_emb_p, ent_transfer_p, rel_transfer_p,
        h_id, t_id, r_id)
    return dist[:b, None], dist[b:, None]


# speedup vs baseline: 1.1375x; 1.1375x over previous
import dataclasses
import functools

import jax
import jax.numpy as jnp
from jax import lax
from jax.experimental import pallas as pl
from jax.experimental.pallas import tpu as pltpu
from jax.experimental.pallas import tpu_sc as plsc

D = 200
DP = 256
NC = 2
NS = 16
L = 16
NW = NC * NS
W = 32
CHUNK_OFFS = tuple(range(0, 13 * L, L))


def _pad_rows(x):
    n = x.shape[0]
    br = 5000 if n % 5000 == 0 else n

    def body(x_ref, o_ref):
        o_ref[...] = jnp.pad(x_ref[...], ((0, 0), (0, DP - D)))

    return pl.pallas_call(
        body,
        grid=(n // br,),
        in_specs=[pl.BlockSpec((br, D), lambda i: (i, 0))],
        out_specs=pl.BlockSpec((br, DP), lambda i: (i, 0)),
        out_shape=jax.ShapeDtypeStruct((n, DP), jnp.float32),
    )(x)


def _pad_rows2(a, b):
    n = a.shape[0]
    br = 5000 if n % 5000 == 0 else n

    def body(a_ref, b_ref, oa_ref, ob_ref):
        oa_ref[...] = jnp.pad(a_ref[...], ((0, 0), (0, DP - D)))
        ob_ref[...] = jnp.pad(b_ref[...], ((0, 0), (0, DP - D)))

    return pl.pallas_call(
        body,
        grid=(n // br,),
        in_specs=[pl.BlockSpec((br, D), lambda i: (i, 0)),
                  pl.BlockSpec((br, D), lambda i: (i, 0))],
        out_specs=[pl.BlockSpec((br, DP), lambda i: (i, 0)),
                   pl.BlockSpec((br, DP), lambda i: (i, 0))],
        out_shape=[jax.ShapeDtypeStruct((n, DP), jnp.float32),
                   jax.ShapeDtypeStruct((n, DP), jnp.float32)],
    )(a, b)


def _rsqrt(x):
    i = lax.bitcast_convert_type(x, jnp.int32)
    i = jnp.int32(0x5F3759DF) - (i >> 1)
    y = lax.bitcast_convert_type(i, jnp.float32)
    for _ in range(3):
        y = y * (jnp.float32(1.5) - jnp.float32(0.5) * x * y * y)
    return y


def _build_dist_kernel(tot):
    rpt = tot // NW
    ch = rpt // W
    mesh = plsc.VectorSubcoreMesh(core_axis_name="c", subcore_axis_name="s")
    cp = pltpu.CompilerParams()
    if "needs_layout_passes" in pltpu.CompilerParams.__dataclass_fields__:
        cp = dataclasses.replace(cp, needs_layout_passes=False)

    @functools.partial(
        pl.kernel,
        mesh=mesh,
        compiler_params=cp,
        out_type=jax.ShapeDtypeStruct((tot,), jnp.float32),
        scratch_types=[
            pltpu.VMEM((rpt,), jnp.int32),
            pltpu.VMEM((rpt,), jnp.int32),
            pltpu.VMEM((rpt,), jnp.int32),
            pltpu.VMEM((2, W, DP), jnp.float32),
            pltpu.VMEM((2, W, DP), jnp.float32),
            pltpu.VMEM((2, W, DP), jnp.float32),
            pltpu.VMEM((2, W, DP), jnp.float32),
            pltpu.VMEM((2, W, DP), jnp.float32),
            pltpu.VMEM((2, W, DP), jnp.float32),
            pltpu.VMEM((rpt,), jnp.float32),
            pltpu.SemaphoreType.DMA,
            pltpu.SemaphoreType.DMA,
        ],
    )
    def dist_kernel(ent_e_hbm, rel_e_hbm, ent_t_hbm, rel_t_hbm,
                    h_hbm, t_hbm, r_hbm, out_hbm,
                    hi, ti, ri, he, ht, te, tt, re, rt, res, sem0, sem1):
        wid = lax.axis_index("s") * NC + lax.axis_index("c")
        base = wid * rpt
        pltpu.sync_copy(h_hbm.at[pl.ds(base, rpt)], hi)
        pltpu.sync_copy(t_hbm.at[pl.ds(base, rpt)], ti)
        pltpu.sync_copy(r_hbm.at[pl.ds(base, rpt)], ri)

        def _issue(off, b, sem):
            return [
                pltpu.async_copy(
                    ent_e_hbm.at[hi.at[pl.ds(off, W)]], he.at[b], sem),
                pltpu.async_copy(
                    ent_t_hbm.at[hi.at[pl.ds(off, W)]], ht.at[b], sem),
                pltpu.async_copy(
                    ent_e_hbm.at[ti.at[pl.ds(off, W)]], te.at[b], sem),
                pltpu.async_copy(
                    ent_t_hbm.at[ti.at[pl.ds(off, W)]], tt.at[b], sem),
                pltpu.async_copy(
                    rel_e_hbm.at[ri.at[pl.ds(off, W)]], re.at[b], sem),
                pltpu.async_copy(
                    rel_t_hbm.at[ri.at[pl.ds(off, W)]], rt.at[b], sem),
            ]

        lane_ids = lax.iota(jnp.int32, L)

        def _compute(off, b):
            def _row(w, dacc):
                z = jnp.zeros((L,), jnp.float32)
                sh = st = ah = at_ = chv = ctv = qv = z
                for o in CHUNK_OFFS:
                    sl = (b, w, pl.ds(o, L))
                    hev, htv = he[sl], ht[sl]
                    tev, ttv = te[sl], tt[sl]
                    rtv = rt[sl]
                    sh = sh + hev * htv
                    st = st + tev * ttv
                    ah = ah + hev * hev
                    at_ = at_ + tev * tev
                    chv = chv + hev * rtv
                    ctv = ctv + tev * rtv
                    qv = qv + rtv * rtv
                s_h, s_t = jnp.sum(sh), jnp.sum(st)
                a_h, a_t = jnp.sum(ah), jnp.sum(at_)
                c_h, c_t = jnp.sum(chv), jnp.sum(ctv)
                q = jnp.sum(qv)

                two = jnp.float32(2.0)
                eps = jnp.float32(1e-12)
                nh = a_h + two * s_h * c_h + s_h * s_h * q
                nt = a_t + two * s_t * c_t + s_t * s_t * q
                ih = _rsqrt(jnp.maximum(nh, eps))
                it = _rsqrt(jnp.maximum(nt, eps))
                g = s_h * ih - s_t * it

                acc = z
                for o in CHUNK_OFFS:
                    sl = (b, w, pl.ds(o, L))
                    hev, tev = he[sl], te[sl]
                    rtv, rev = rt[sl], re[sl]
                    term = jnp.abs(ih * hev - it * tev + g * rtv + rev)
                    acc = acc + term
                return jnp.where(lane_ids == w % L, jnp.sum(acc), dacc)

            for sub in range(W // L):
                dvec = lax.fori_loop(
                    0, L,
                    lambda w, a, _s=sub: _row(jnp.int32(_s * L) + w, a),
                    jnp.zeros((L,), jnp.float32))
                res[pl.ds(off + sub * L, L)] = dvec

        def _drain(b, sem):
            for dst in (he, ht, te, tt, re, rt):
                pltpu.make_async_copy(
                    ent_e_hbm.at[pl.ds(0, W)], dst.at[b], sem).wait()

        _issue(0, 0, sem0)
        last = rpt - W

        @pl.loop(0, ch // 2)
        def _pair(p):
            off0 = pl.multiple_of(p * (2 * W), W)
            off1 = off0 + W
            _issue(off1, 1, sem1)
            _drain(0, sem0)
            _compute(off0, 0)
            off2 = jnp.minimum(off0 + 2 * W, jnp.int32(last))
            _issue(off2, 0, sem0)
            _drain(1, sem1)
            _compute(off1, 1)

        _drain(0, sem0)
        pltpu.sync_copy(res, out_hbm.at[pl.ds(base, rpt)])

    return dist_kernel


def kernel(ent_emb, rel_emb, ent_transfer, rel_transfer,
           pos_h_id, pos_t_id, pos_r_id, neg_h_id, neg_t_id, neg_r_id):
    b = pos_h_id.shape[0]
    h_id = jnp.concatenate([pos_h_id, neg_h_id]).astype(jnp.int32)
    t_id = jnp.concatenate([pos_t_id, neg_t_id]).astype(jnp.int32)
    r_id = jnp.concatenate([pos_r_id, neg_r_id]).astype(jnp.int32)
    ent_emb_p, ent_transfer_p = _pad_rows2(ent_emb, ent_transfer)
    rel_emb_p, rel_transfer_p = _pad_rows2(rel_emb, rel_transfer)
    dist = _build_dist_kernel(2 * b)(
        ent_emb_p, rel_emb_p, ent_transfer_p, rel_transfer_p,
        h_id, t_id, r_id)
    return dist[:b, None], dist[b:, None]


# speedup vs baseline: 1.1381x; 1.0006x over previous
import dataclasses
import functools

import jax
import jax.numpy as jnp
from jax import lax
from jax.experimental import pallas as pl
from jax.experimental.pallas import tpu as pltpu
from jax.experimental.pallas import tpu_sc as plsc

D = 200
DP = 256
NC = 2
NS = 16
L = 16
NW = NC * NS
W = 32
CHUNK_OFFS = tuple(range(0, 13 * L, L))


def _pad_rows2(a, b):
    n = a.shape[0]
    br = 5000 if n % 5000 == 0 else n

    def body(a_ref, b_ref, oa_ref, ob_ref):
        oa_ref[...] = jnp.pad(a_ref[...], ((0, 0), (0, DP - D)))
        ob_ref[...] = jnp.pad(b_ref[...], ((0, 0), (0, DP - D)))

    return pl.pallas_call(
        body,
        grid=(n // br,),
        in_specs=[pl.BlockSpec((br, D), lambda i: (i, 0)),
                  pl.BlockSpec((br, D), lambda i: (i, 0))],
        out_specs=[pl.BlockSpec((br, DP), lambda i: (i, 0)),
                   pl.BlockSpec((br, DP), lambda i: (i, 0))],
        out_shape=[jax.ShapeDtypeStruct((n, DP), jnp.float32),
                   jax.ShapeDtypeStruct((n, DP), jnp.float32)],
    )(a, b)


def _rsqrt(x):
    i = lax.bitcast_convert_type(x, jnp.int32)
    i = jnp.int32(0x5F3759DF) - (i >> 1)
    y = lax.bitcast_convert_type(i, jnp.float32)
    for _ in range(3):
        y = y * (jnp.float32(1.5) - jnp.float32(0.5) * x * y * y)
    return y


def _build_dist_kernel(tot):
    rpt = tot // NW
    ch = rpt // W
    mesh = plsc.VectorSubcoreMesh(core_axis_name="c", subcore_axis_name="s")
    cp = pltpu.CompilerParams()
    if "needs_layout_passes" in pltpu.CompilerParams.__dataclass_fields__:
        cp = dataclasses.replace(cp, needs_layout_passes=False)

    @functools.partial(
        pl.kernel,
        mesh=mesh,
        compiler_params=cp,
        out_type=jax.ShapeDtypeStruct((tot,), jnp.float32),
        scratch_types=[
            pltpu.VMEM((rpt,), jnp.int32),
            pltpu.VMEM((rpt,), jnp.int32),
            pltpu.VMEM((rpt,), jnp.int32),
            pltpu.VMEM((2, W, DP), jnp.float32),
            pltpu.VMEM((2, W, DP), jnp.float32),
            pltpu.VMEM((2, W, DP), jnp.float32),
            pltpu.VMEM((2, W, DP), jnp.float32),
            pltpu.VMEM((2, W, DP), jnp.float32),
            pltpu.VMEM((2, W, DP), jnp.float32),
            pltpu.VMEM((rpt,), jnp.float32),
            pltpu.SemaphoreType.DMA,
            pltpu.SemaphoreType.DMA,
        ],
    )
    def dist_kernel(ent_e_hbm, rel_e_hbm, ent_t_hbm, rel_t_hbm,
                    h_hbm, t_hbm, r_hbm, out_hbm,
                    hi, ti, ri, he, ht, te, tt, re, rt, res, sem0, sem1):
        wid = lax.axis_index("s") * NC + lax.axis_index("c")
        base = wid * rpt
        pltpu.sync_copy(h_hbm.at[pl.ds(base, rpt)], hi)
        pltpu.sync_copy(t_hbm.at[pl.ds(base, rpt)], ti)
        pltpu.sync_copy(r_hbm.at[pl.ds(base, rpt)], ri)

        def _issue(off, b, sem):
            return [
                pltpu.async_copy(
                    ent_e_hbm.at[hi.at[pl.ds(off, W)]], he.at[b], sem),
                pltpu.async_copy(
                    ent_t_hbm.at[hi.at[pl.ds(off, W)]], ht.at[b], sem),
                pltpu.async_copy(
                    ent_e_hbm.at[ti.at[pl.ds(off, W)]], te.at[b], sem),
                pltpu.async_copy(
                    ent_t_hbm.at[ti.at[pl.ds(off, W)]], tt.at[b], sem),
                pltpu.async_copy(
                    rel_e_hbm.at[ri.at[pl.ds(off, W)]], re.at[b], sem),
                pltpu.async_copy(
                    rel_t_hbm.at[ri.at[pl.ds(off, W)]], rt.at[b], sem),
            ]

        lane_ids = lax.iota(jnp.int32, L)

        def _compute(off, b):
            def _row(w, dacc):
                z = jnp.zeros((L,), jnp.float32)
                sh = st = ah = at_ = chv = ctv = qv = z
                for o in CHUNK_OFFS:
                    sl = (b, w, pl.ds(o, L))
                    hev, htv = he[sl], ht[sl]
                    tev, ttv = te[sl], tt[sl]
                    rtv = rt[sl]
                    sh = sh + hev * htv
                    st = st + tev * ttv
                    ah = ah + hev * hev
                    at_ = at_ + tev * tev
                    chv = chv + hev * rtv
                    ctv = ctv + tev * rtv
                    qv = qv + rtv * rtv
                s_h, s_t = jnp.sum(sh), jnp.sum(st)
                a_h, a_t = jnp.sum(ah), jnp.sum(at_)
                c_h, c_t = jnp.sum(chv), jnp.sum(ctv)
                q = jnp.sum(qv)

                two = jnp.float32(2.0)
                eps = jnp.float32(1e-12)
                nh = a_h + two * s_h * c_h + s_h * s_h * q
                nt = a_t + two * s_t * c_t + s_t * s_t * q
                ih = _rsqrt(jnp.maximum(nh, eps))
                it = _rsqrt(jnp.maximum(nt, eps))
                g = s_h * ih - s_t * it

                acc = z
                for o in CHUNK_OFFS:
                    sl = (b, w, pl.ds(o, L))
                    hev, tev = he[sl], te[sl]
                    rtv, rev = rt[sl], re[sl]
                    term = jnp.abs(ih * hev - it * tev + g * rtv + rev)
                    acc = acc + term
                return jnp.where(lane_ids == w % L, jnp.sum(acc), dacc)

            for sub in range(W // L):
                dvec = lax.fori_loop(
                    0, L,
                    lambda w, a, _s=sub: _row(jnp.int32(_s * L) + w, a),
                    jnp.zeros((L,), jnp.float32))
                res[pl.ds(off + sub * L, L)] = dvec

        def _drain(b, sem):
            for dst in (he, ht, te, tt, re, rt):
                pltpu.make_async_copy(
                    ent_e_hbm.at[pl.ds(0, W)], dst.at[b], sem).wait()

        _issue(0, 0, sem0)
        last = rpt - W

        @pl.loop(0, ch // 2)
        def _pair(p):
            off0 = pl.multiple_of(p * (2 * W), W)
            off1 = off0 + W
            _issue(off1, 1, sem1)
            _drain(0, sem0)
            _compute(off0, 0)
            off2 = jnp.minimum(off0 + 2 * W, jnp.int32(last))
            _issue(off2, 0, sem0)
            _drain(1, sem1)
            _compute(off1, 1)

        _drain(0, sem0)
        pltpu.sync_copy(res, out_hbm.at[pl.ds(base, rpt)])

    return dist_kernel


def kernel(ent_emb, rel_emb, ent_transfer, rel_transfer,
           pos_h_id, pos_t_id, pos_r_id, neg_h_id, neg_t_id, neg_r_id):
    b = pos_h_id.shape[0]
    h_id = jnp.concatenate([pos_h_id, neg_h_id]).astype(jnp.int32)
    t_id = jnp.concatenate([pos_t_id, neg_t_id]).astype(jnp.int32)
    r_id = jnp.concatenate([pos_r_id, neg_r_id]).astype(jnp.int32)
    ent_emb_p, ent_transfer_p = _pad_rows2(ent_emb, ent_transfer)
    rel_emb_p, rel_transfer_p = _pad_rows2(rel_emb, rel_transfer)
    dist = _build_dist_kernel(2 * b)(
        ent_emb_p, rel_emb_p, ent_transfer_p, rel_transfer_p,
        h_id, t_id, r_id)
    return dist[:b, None], dist[b:, None]
